# baseline (speedup 1.0000x reference)
import functools

import jax
import jax.numpy as jnp
from jax import lax
from jax.experimental import pallas as pl
from jax.experimental.pallas import tpu as pltpu
from jax.experimental.pallas import tpu_sc as plsc

_LANES = 128
_ROW_BLK = 256
_COL_BLK = 2048
_BIG_IDX = 2**30


def _argmin_body(x_ref, out_ref, vacc, iacc, *, n_cols):
    j = pl.program_id(1)
    nb = pl.num_programs(1)
    n_chunks = _COL_BLK // _LANES

    @pl.when(j == 0)
    def _init():
        vacc[...] = jnp.full(vacc.shape, jnp.inf, dtype=vacc.dtype)
        iacc[...] = jnp.zeros(iacc.shape, dtype=iacc.dtype)

    lane = lax.broadcasted_iota(jnp.int32, vacc.shape, 1)

    def update(k, masked):
        xk = x_ref[:, k * _LANES:(k + 1) * _LANES]
        idx = lane + (j * _COL_BLK + k * _LANES)
        if masked:
            xk = jnp.where(idx < n_cols, xk, jnp.inf)
        cmp = xk < vacc[...]
        vacc[...] = jnp.where(cmp, xk, vacc[...])
        iacc[...] = jnp.where(cmp, idx, iacc[...])

    @pl.when(j < nb - 1)
    def _full():
        for k in range(n_chunks):
            update(k, False)

    @pl.when(j == nb - 1)
    def _tail():
        for k in range(n_chunks):
            update(k, True)
        v = vacc[...]
        gmin = jnp.min(v, axis=1, keepdims=True)
        cand = jnp.where(v == gmin, iacc[...], _BIG_IDX)
        out_ref[...] = jnp.min(cand, axis=1, keepdims=True)


def _argmin_rows(distances):
    n_rows, n_cols = distances.shape
    n_rb = n_rows // _ROW_BLK
    n_cb = -(-n_cols // _COL_BLK)
    out = pl.pallas_call(
        functools.partial(_argmin_body, n_cols=n_cols),
        grid=(n_rb, n_cb),
        in_specs=[pl.BlockSpec((_ROW_BLK, _COL_BLK), lambda r, j: (r, j))],
        out_specs=pl.BlockSpec((_ROW_BLK, 1), lambda r, j: (r, 0)),
        out_shape=jax.ShapeDtypeStruct((n_rows, 1), jnp.int32),
        scratch_shapes=[
            pltpu.VMEM((_ROW_BLK, _LANES), jnp.float32),
            pltpu.VMEM((_ROW_BLK, _LANES), jnp.int32),
        ],
        compiler_params=pltpu.CompilerParams(
            dimension_semantics=("arbitrary", "arbitrary"),
        ),
    )(distances)
    return out.reshape(n_rows)


def _label_gather(labels, win_idx):
    info = plsc.get_sparse_core_info()
    n_workers = info.num_cores * info.num_subcores
    b = win_idx.shape[0]
    b_per_w = b // n_workers
    mesh = plsc.VectorSubcoreMesh(core_axis_name="c", subcore_axis_name="s")

    @functools.partial(
        pl.kernel,
        mesh=mesh,
        out_type=jax.ShapeDtypeStruct((b,), labels.dtype),
        scratch_types=[
            pltpu.VMEM((b_per_w,), jnp.int32),
            pltpu.VMEM((b_per_w,), jnp.int32),
            pltpu.SemaphoreType.DMA,
        ],
    )
    def gather_kernel(labels_hbm, idx_hbm, out_hbm, idx_v, out_v, sem):
        wid = lax.axis_index("s") * info.num_cores + lax.axis_index("c")
        base = wid * b_per_w
        pltpu.sync_copy(idx_hbm.at[pl.ds(base, b_per_w)], idx_v)
        pltpu.async_copy(labels_hbm.at[idx_v], out_v, sem).wait()
        pltpu.sync_copy(out_v, out_hbm.at[pl.ds(base, b_per_w)])

    return gather_kernel(labels, win_idx)


def kernel(distances, labels):
    win_idx = _argmin_rows(distances)
    return _label_gather(labels, win_idx)


# speedup vs baseline: 1.0323x; 1.0323x over previous
import functools

import jax
import jax.numpy as jnp
from jax import lax
from jax.experimental import pallas as pl
from jax.experimental.pallas import tpu as pltpu
from jax.experimental.pallas import tpu_sc as plsc

_LANES = 128
_ROW_BLK = 256
_COL_BLK = 2048
_BIG_IDX = 2**30


def _argmin_body(x_ref, out_ref, vacc, iacc, *, n_cols):
    j = pl.program_id(1)
    nb = pl.num_programs(1)
    n_chunks = _COL_BLK // _LANES
    n_groups = _ROW_BLK // 8

    @pl.when(j == 0)
    def _init():
        vacc[...] = jnp.full(vacc.shape, jnp.inf, dtype=vacc.dtype)
        iacc[...] = jnp.zeros(iacc.shape, dtype=iacc.dtype)

    def sweep(masked):
        for g in range(n_groups):
            rows = pl.ds(g * 8, 8)
            v = vacc[rows, :]
            i = iacc[rows, :]
            for k in range(n_chunks):
                xk = x_ref[rows, k * _LANES:(k + 1) * _LANES]
                c = j * n_chunks + k
                if masked:
                    lane = lax.broadcasted_iota(jnp.int32, xk.shape, 1)
                    xk = jnp.where(lane + c * _LANES < n_cols, xk, jnp.inf)
                cmp = xk < v
                v = jnp.minimum(v, xk)
                i = jnp.where(cmp, c, i)
            vacc[rows, :] = v
            iacc[rows, :] = i

    @pl.when(j < nb - 1)
    def _full():
        sweep(False)

    @pl.when(j == nb - 1)
    def _tail():
        sweep(True)
        v = vacc[...]
        lane = lax.broadcasted_iota(jnp.int32, v.shape, 1)
        gidx = iacc[...] * _LANES + lane
        gmin = jnp.min(v, axis=1, keepdims=True)
        cand = jnp.where(v == gmin, gidx, _BIG_IDX)
        out_ref[...] = jnp.min(cand, axis=1, keepdims=True)


def _argmin_rows(distances):
    n_rows, n_cols = distances.shape
    n_rb = n_rows // _ROW_BLK
    n_cb = -(-n_cols // _COL_BLK)
    out = pl.pallas_call(
        functools.partial(_argmin_body, n_cols=n_cols),
        grid=(n_rb, n_cb),
        in_specs=[pl.BlockSpec((_ROW_BLK, _COL_BLK), lambda r, j: (r, j))],
        out_specs=pl.BlockSpec((_ROW_BLK, 1), lambda r, j: (r, 0)),
        out_shape=jax.ShapeDtypeStruct((n_rows, 1), jnp.int32),
        scratch_shapes=[
            pltpu.VMEM((_ROW_BLK, _LANES), jnp.float32),
            pltpu.VMEM((_ROW_BLK, _LANES), jnp.int32),
        ],
        compiler_params=pltpu.CompilerParams(
            dimension_semantics=("arbitrary", "arbitrary"),
        ),
    )(distances)
    return out.reshape(n_rows)


def _label_gather(labels, win_idx):
    info = plsc.get_sparse_core_info()
    n_workers = info.num_cores * info.num_subcores
    b = win_idx.shape[0]
    b_per_w = b // n_workers
    mesh = plsc.VectorSubcoreMesh(core_axis_name="c", subcore_axis_name="s")

    @functools.partial(
        pl.kernel,
        mesh=mesh,
        out_type=jax.ShapeDtypeStruct((b,), labels.dtype),
        scratch_types=[
            pltpu.VMEM((b_per_w,), jnp.int32),
            pltpu.VMEM((b_per_w,), jnp.int32),
            pltpu.SemaphoreType.DMA,
        ],
    )
    def gather_kernel(labels_hbm, idx_hbm, out_hbm, idx_v, out_v, sem):
        wid = lax.axis_index("s") * info.num_cores + lax.axis_index("c")
        base = wid * b_per_w
        pltpu.sync_copy(idx_hbm.at[pl.ds(base, b_per_w)], idx_v)
        pltpu.async_copy(labels_hbm.at[idx_v], out_v, sem).wait()
        pltpu.sync_copy(out_v, out_hbm.at[pl.ds(base, b_per_w)])

    return gather_kernel(labels, win_idx)


def kernel(distances, labels):
    win_idx = _argmin_rows(distances)
    return _label_gather(labels, win_idx)


# speedup vs baseline: 3.7800x; 3.6617x over previous
import functools

import jax
import jax.numpy as jnp
from jax import lax
from jax.experimental import pallas as pl
from jax.experimental.pallas import tpu as pltpu
from jax.experimental.pallas import tpu_sc as plsc

_ROW_BLK = 2000
_BIG_IDX = 2**30


def _argmin_body(x_ref, out_ref, vacc, iacc):
    j = pl.program_id(0)
    nb = pl.num_programs(0)
    n_groups = _ROW_BLK // 8

    @pl.when(j == 0)
    def _init():
        vacc[...] = jnp.full(vacc.shape, jnp.inf, dtype=vacc.dtype)
        iacc[...] = jnp.zeros(iacc.shape, dtype=iacc.dtype)

    v = vacc[...]
    i = iacc[...]
    for g in range(n_groups):
        xg = x_ref[pl.ds(g * 8, 8), :]
        c = j * n_groups + g
        cmp = xg < v
        v = jnp.minimum(v, xg)
        i = jnp.where(cmp, c, i)
    vacc[...] = v
    iacc[...] = i

    @pl.when(j == nb - 1)
    def _merge():
        vf = vacc[...]
        sub = lax.broadcasted_iota(jnp.int32, vf.shape, 0)
        gidx = iacc[...] * 8 + sub
        gmin = jnp.min(vf, axis=0, keepdims=True)
        cand = jnp.where(vf == gmin, gidx, _BIG_IDX)
        out_ref[...] = jnp.min(cand, axis=0, keepdims=True)


def _argmin_cols(xt):
    n_protos, n_samples = xt.shape
    nb = n_protos // _ROW_BLK
    out = pl.pallas_call(
        _argmin_body,
        grid=(nb,),
        in_specs=[pl.BlockSpec((_ROW_BLK, n_samples), lambda j: (j, 0))],
        out_specs=pl.BlockSpec((1, n_samples), lambda j: (0, 0)),
        out_shape=jax.ShapeDtypeStruct((1, n_samples), jnp.int32),
        scratch_shapes=[
            pltpu.VMEM((8, n_samples), jnp.float32),
            pltpu.VMEM((8, n_samples), jnp.int32),
        ],
        compiler_params=pltpu.CompilerParams(
            dimension_semantics=("arbitrary",),
        ),
    )(xt)
    return out.reshape(n_samples)


def _label_gather(labels, win_idx):
    info = plsc.get_sparse_core_info()
    n_workers = info.num_cores * info.num_subcores
    b = win_idx.shape[0]
    b_per_w = b // n_workers
    mesh = plsc.VectorSubcoreMesh(core_axis_name="c", subcore_axis_name="s")

    @functools.partial(
        pl.kernel,
        mesh=mesh,
        out_type=jax.ShapeDtypeStruct((b,), labels.dtype),
        scratch_types=[
            pltpu.VMEM((b_per_w,), jnp.int32),
            pltpu.VMEM((b_per_w,), jnp.int32),
            pltpu.SemaphoreType.DMA,
        ],
    )
    def gather_kernel(labels_hbm, idx_hbm, out_hbm, idx_v, out_v, sem):
        wid = lax.axis_index("s") * info.num_cores + lax.axis_index("c")
        base = wid * b_per_w
        pltpu.sync_copy(idx_hbm.at[pl.ds(base, b_per_w)], idx_v)
        pltpu.async_copy(labels_hbm.at[idx_v], out_v, sem).wait()
        pltpu.sync_copy(out_v, out_hbm.at[pl.ds(base, b_per_w)])

    return gather_kernel(labels, win_idx)


def kernel(distances, labels):
    win_idx = _argmin_cols(distances.T)
    return _label_gather(labels, win_idx)


# speedup vs baseline: 4.0079x; 1.0603x over previous
import functools

import jax
import jax.numpy as jnp
from jax import lax
from jax.experimental import pallas as pl
from jax.experimental.pallas import tpu as pltpu
from jax.experimental.pallas import tpu_sc as plsc

_ROW_BLK = 2000
_BIG_IDX = 2**30


def _argmin_body(x_ref, out_ref, vacc, iacc):
    j = pl.program_id(0)
    nb = pl.num_programs(0)
    n_groups = _ROW_BLK // 8
    n_lgrp = vacc.shape[1] // 128

    @pl.when(j == 0)
    def _init():
        vacc[...] = jnp.full(vacc.shape, jnp.inf, dtype=vacc.dtype)
        iacc[...] = jnp.zeros(iacc.shape, dtype=iacc.dtype)

    for l in range(n_lgrp):
        lanes = pl.ds(l * 128, 128)
        v = vacc[:, lanes]
        i = iacc[:, lanes]
        for g in range(n_groups):
            xg = x_ref[pl.ds(g * 8, 8), lanes]
            cmp = xg < v
            v = jnp.minimum(v, xg)
            i = jnp.where(cmp, j * n_groups + g, i)
        vacc[:, lanes] = v
        iacc[:, lanes] = i

    @pl.when(j == nb - 1)
    def _merge():
        vf = vacc[...]
        sub = lax.broadcasted_iota(jnp.int32, vf.shape, 0)
        gidx = iacc[...] * 8 + sub
        gmin = jnp.min(vf, axis=0, keepdims=True)
        cand = jnp.where(vf == gmin, gidx, _BIG_IDX)
        out_ref[...] = jnp.min(cand, axis=0, keepdims=True)


def _argmin_cols(xt):
    n_protos, n_samples = xt.shape
    nb = n_protos // _ROW_BLK
    out = pl.pallas_call(
        _argmin_body,
        grid=(nb,),
        in_specs=[pl.BlockSpec((_ROW_BLK, n_samples), lambda j: (j, 0))],
        out_specs=pl.BlockSpec((1, n_samples), lambda j: (0, 0)),
        out_shape=jax.ShapeDtypeStruct((1, n_samples), jnp.int32),
        scratch_shapes=[
            pltpu.VMEM((8, n_samples), jnp.float32),
            pltpu.VMEM((8, n_samples), jnp.int32),
        ],
        compiler_params=pltpu.CompilerParams(
            dimension_semantics=("arbitrary",),
        ),
    )(xt)
    return out.reshape(n_samples)


def _label_gather(labels, win_idx):
    info = plsc.get_sparse_core_info()
    n_workers = info.num_cores * info.num_subcores
    b = win_idx.shape[0]
    b_per_w = b // n_workers
    mesh = plsc.VectorSubcoreMesh(core_axis_name="c", subcore_axis_name="s")

    @functools.partial(
        pl.kernel,
        mesh=mesh,
        out_type=jax.ShapeDtypeStruct((b,), labels.dtype),
        scratch_types=[
            pltpu.VMEM((b_per_w,), jnp.int32),
            pltpu.VMEM((b_per_w,), jnp.int32),
            pltpu.SemaphoreType.DMA,
        ],
    )
    def gather_kernel(labels_hbm, idx_hbm, out_hbm, idx_v, out_v, sem):
        wid = lax.axis_index("s") * info.num_cores + lax.axis_index("c")
        base = wid * b_per_w
        pltpu.sync_copy(idx_hbm.at[pl.ds(base, b_per_w)], idx_v)
        pltpu.async_copy(labels_hbm.at[idx_v], out_v, sem).wait()
        pltpu.sync_copy(out_v, out_hbm.at[pl.ds(base, b_per_w)])

    return gather_kernel(labels, win_idx)


def kernel(distances, labels):
    win_idx = _argmin_cols(distances.T)
    return _label_gather(labels, win_idx)
